# baseline (speedup 1.0000x reference)
import functools

import jax
import jax.numpy as jnp
from jax import lax
from jax.experimental import pallas as pl
from jax.experimental.pallas import tpu as pltpu
from jax.experimental.pallas import tpu_sc as plsc

_B = 16384
_D = 2048
_C = 10
_B_SC = 3072
_B_TC = _B - _B_SC
_N_TILES = 32
_ROWS_PER_TILE = _B_SC // _N_TILES
_CHUNK = 16
_LANES = 16


def _tc_kernel(f_ref, w_ref, b_ref, o_ref):
    acc = lax.dot_general(
        f_ref[...].astype(jnp.bfloat16), w_ref[...].astype(jnp.bfloat16),
        dimension_numbers=(((1,), (1,)), ((), ())),
        preferred_element_type=jnp.float32,
    )
    o_ref[...] = acc + b_ref[...]


def _tc_part(feat, W, b2):
    Bm = 1024
    return pl.pallas_call(
        _tc_kernel,
        grid=(_B_TC // Bm,),
        in_specs=[
            pl.BlockSpec((Bm, _D), lambda i: (i, 0)),
            pl.BlockSpec((_C, _D), lambda i: (0, 0)),
            pl.BlockSpec((1, _C), lambda i: (0, 0)),
        ],
        out_specs=pl.BlockSpec((Bm, _C), lambda i: (i, 0)),
        out_shape=jax.ShapeDtypeStruct((_B_TC, _C), jnp.float32),
        compiler_params=pltpu.CompilerParams(
            dimension_semantics=("arbitrary",),
        ),
    )(feat, W, b2)


def _sc_body(feat_hbm, w_hbm, b_hbm, out_hbm, wv, bv, fch, ov):
    wid = lax.axis_index("s") * 2 + lax.axis_index("c")
    base = wid * _ROWS_PER_TILE

    pltpu.sync_copy(w_hbm, wv)
    pltpu.sync_copy(b_hbm, bv)
    bias = bv[...]
    lanes = lax.iota(jnp.int32, _LANES)
    zero = jnp.zeros((_LANES,), jnp.float32)

    def chunk_body(ch, carry):
        row0 = base + ch * _CHUNK
        pltpu.sync_copy(feat_hbm.at[pl.ds(_B_TC + row0, _CHUNK), :], fch)

        for p in range(_CHUNK // 2):
            r0, r1 = 2 * p, 2 * p + 1

            def d_body(d, accs, _r0=r0, _r1=r1):
                sl = pl.ds(d * _LANES, _LANES)
                f0 = fch[_r0, sl]
                f1 = fch[_r1, sl]
                new = []
                for c in range(_C):
                    wc = wv[c, sl]
                    new.append(accs[c] + f0 * wc)
                    new.append(accs[_C + c] + f1 * wc)
                return tuple(new[0::2]) + tuple(new[1::2])

            init = tuple(jnp.zeros((_LANES,), jnp.float32)
                         for _ in range(2 * _C))
            accs = lax.fori_loop(0, _D // _LANES, d_body, init, unroll=2)

            res0 = bias
            res1 = bias
            for c in range(_C):
                s0 = jnp.sum(accs[c])
                s1 = jnp.sum(accs[_C + c])
                m = lanes == c
                res0 = res0 + jnp.where(m, jnp.full((_LANES,), s0), zero)
                res1 = res1 + jnp.where(m, jnp.full((_LANES,), s1), zero)
            ov[r0] = res0
            ov[r1] = res1

        pltpu.sync_copy(ov, out_hbm.at[pl.ds(row0, _CHUNK), :])
        return carry

    lax.fori_loop(0, _ROWS_PER_TILE // _CHUNK, chunk_body, 0)


_sc_part = pl.kernel(
    _sc_body,
    out_type=jax.ShapeDtypeStruct((_B_SC, _LANES), jnp.float32),
    mesh=plsc.VectorSubcoreMesh(core_axis_name="c", subcore_axis_name="s"),
    scratch_types=[
        pltpu.VMEM((_C, _D), jnp.float32),
        pltpu.VMEM((_LANES,), jnp.float32),
        pltpu.VMEM((_CHUNK, _D), jnp.float32),
        pltpu.VMEM((_CHUNK, _LANES), jnp.float32),
    ],
    compiler_params=pltpu.CompilerParams(needs_layout_passes=False),
)


def kernel(feat, W, b):
    b2 = b.reshape(1, _C)
    b16 = jnp.pad(b, (0, _LANES - _C))
    tc_out = _tc_part(feat, W, b2)
    sc_out = _sc_part(feat, W, b16)
    return jnp.concatenate([tc_out, sc_out[:, :_C]], axis=0)
